# baseline (speedup 1.0000x reference)
import jax
import jax.numpy as jnp
from jax.experimental import pallas as pl
from jax.experimental.pallas import tpu as pltpu

_BM = 2000


def _chain_kernel(x_ref, hm_ref, hp_ref, wcat_ref, w1_ref, w2_ref, wfc_ref,
                  bt_ref, b1_ref, bf_ref, out_ref):
    cur = x_ref[...]
    bm = cur.shape[0]
    rowid = jax.lax.broadcasted_iota(jnp.int32, cur.shape, 0)
    xm1 = pltpu.roll(cur, shift=1, axis=0)
    xm1 = jnp.where(rowid == 0, hm_ref[0], xm1)
    xp1 = pltpu.roll(cur, shift=bm - 1, axis=0)
    xp1 = jnp.where(rowid == bm - 1, hp_ref[0], xp1)
    xin = jnp.concatenate([xm1, cur, xp1], axis=1)
    y = jnp.dot(xin, wcat_ref[...], preferred_element_type=jnp.float32)
    y = jax.nn.relu(y + bt_ref[...])
    z = jnp.dot(y, w1_ref[...], preferred_element_type=jnp.float32)
    z = jax.nn.relu(z + b1_ref[...])
    wf = jnp.dot(w2_ref[...], wfc_ref[...], preferred_element_type=jnp.float32)
    z = jnp.dot(z, wf, preferred_element_type=jnp.float32) + bf_ref[...]
    out_ref[...] = z


def kernel(x, edge_index, edge_weights, W_t, b_t, W1, b1, W2, b2, W_fc, b_fc):
    B, T, N, C = x.shape
    H = W1.shape[0]
    C_OUT = W_fc.shape[1]
    rows = B * N
    bm = _BM
    nblk = rows // bm

    Wcat = jnp.concatenate(
        [W_t[:, :, 0, 0].T, W_t[:, :, 1, 0].T, W_t[:, :, 2, 0].T], axis=0)
    bf = (b2 @ W_fc + b_fc).reshape(1, C_OUT)

    xl = x[:, T - 1].reshape(rows, C)

    starts = jnp.arange(nblk, dtype=jnp.int32) * bm
    ends = starts + bm
    hm = jnp.where(((starts % N) != 0)[:, None],
                   xl[jnp.clip(starts - 1, 0, rows - 1)], 0.0).reshape(nblk, 1, C)
    hp = jnp.where(((ends % N) != 0)[:, None],
                   xl[jnp.clip(ends, 0, rows - 1)], 0.0).reshape(nblk, 1, C)

    out = pl.pallas_call(
        _chain_kernel,
        grid=(nblk,),
        in_specs=[
            pl.BlockSpec((bm, C), lambda k: (k, 0)),
            pl.BlockSpec((1, 1, C), lambda k: (k, 0, 0)),
            pl.BlockSpec((1, 1, C), lambda k: (k, 0, 0)),
            pl.BlockSpec((3 * C, H), lambda k: (0, 0)),
            pl.BlockSpec((H, H), lambda k: (0, 0)),
            pl.BlockSpec((H, H), lambda k: (0, 0)),
            pl.BlockSpec((H, C_OUT), lambda k: (0, 0)),
            pl.BlockSpec((1, H), lambda k: (0, 0)),
            pl.BlockSpec((1, H), lambda k: (0, 0)),
            pl.BlockSpec((1, C_OUT), lambda k: (0, 0)),
        ],
        out_specs=pl.BlockSpec((bm, C_OUT), lambda k: (k, 0)),
        out_shape=jax.ShapeDtypeStruct((rows, C_OUT), jnp.float32),
    )(xl, hm, hp, Wcat, W1, W2, W_fc,
      b_t.reshape(1, H), b1.reshape(1, H), bf)
    return out.reshape(B, N, C_OUT)


# speedup vs baseline: 1.0796x; 1.0796x over previous
import jax
import jax.numpy as jnp
from jax.experimental import pallas as pl
from jax.experimental.pallas import tpu as pltpu

_BM = 5000


def _chain_kernel(x_ref, hm_ref, hp_ref, wcat_ref, w1_ref, w2_ref, wfc_ref,
                  bt_ref, b1_ref, bf_ref, out_ref):
    cur = x_ref[...]
    bm = cur.shape[0]
    rowid = jax.lax.broadcasted_iota(jnp.int32, cur.shape, 0)
    xm1 = pltpu.roll(cur, shift=1, axis=0)
    xm1 = jnp.where(rowid == 0, hm_ref[0], xm1)
    xp1 = pltpu.roll(cur, shift=bm - 1, axis=0)
    xp1 = jnp.where(rowid == bm - 1, hp_ref[0], xp1)
    xin = jnp.concatenate([xm1, cur, xp1], axis=1)
    y = jnp.dot(xin, wcat_ref[...], preferred_element_type=jnp.float32)
    y = jax.nn.relu(y + bt_ref[...])
    z = jnp.dot(y, w1_ref[...], preferred_element_type=jnp.float32)
    z = jax.nn.relu(z + b1_ref[...])
    wf = jnp.dot(w2_ref[...], wfc_ref[...], preferred_element_type=jnp.float32)
    z = jnp.dot(z, wf, preferred_element_type=jnp.float32) + bf_ref[...]
    out_ref[...] = z


def kernel(x, edge_index, edge_weights, W_t, b_t, W1, b1, W2, b2, W_fc, b_fc):
    B, T, N, C = x.shape
    H = W1.shape[0]
    C_OUT = W_fc.shape[1]
    rows = B * N
    bm = _BM
    nblk = rows // bm

    Wcat = jnp.concatenate(
        [W_t[:, :, 0, 0].T, W_t[:, :, 1, 0].T, W_t[:, :, 2, 0].T], axis=0)
    bf = (b2 @ W_fc + b_fc).reshape(1, C_OUT)

    xl = x[:, T - 1].reshape(rows, C)

    starts = jnp.arange(nblk, dtype=jnp.int32) * bm
    ends = starts + bm
    hm = jnp.where(((starts % N) != 0)[:, None],
                   xl[jnp.clip(starts - 1, 0, rows - 1)], 0.0).reshape(nblk, 1, C)
    hp = jnp.where(((ends % N) != 0)[:, None],
                   xl[jnp.clip(ends, 0, rows - 1)], 0.0).reshape(nblk, 1, C)

    out = pl.pallas_call(
        _chain_kernel,
        grid=(nblk,),
        in_specs=[
            pl.BlockSpec((bm, C), lambda k: (k, 0)),
            pl.BlockSpec((1, 1, C), lambda k: (k, 0, 0)),
            pl.BlockSpec((1, 1, C), lambda k: (k, 0, 0)),
            pl.BlockSpec((3 * C, H), lambda k: (0, 0)),
            pl.BlockSpec((H, H), lambda k: (0, 0)),
            pl.BlockSpec((H, H), lambda k: (0, 0)),
            pl.BlockSpec((H, C_OUT), lambda k: (0, 0)),
            pl.BlockSpec((1, H), lambda k: (0, 0)),
            pl.BlockSpec((1, H), lambda k: (0, 0)),
            pl.BlockSpec((1, C_OUT), lambda k: (0, 0)),
        ],
        out_specs=pl.BlockSpec((bm, C_OUT), lambda k: (k, 0)),
        out_shape=jax.ShapeDtypeStruct((rows, C_OUT), jnp.float32),
    )(xl, hm, hp, Wcat, W1, W2, W_fc,
      b_t.reshape(1, H), b1.reshape(1, H), bf)
    return out.reshape(B, N, C_OUT)


# speedup vs baseline: 1.2959x; 1.2004x over previous
import jax
import jax.numpy as jnp
from jax.experimental import pallas as pl
from jax.experimental.pallas import tpu as pltpu


def _chain_kernel(x_ref, w0_ref, w1t_ref, w2t_ref, w1_ref, w2_ref, wfc_ref,
                  bt_ref, b1_ref, bf_ref, out_ref):
    cur = x_ref[...]
    n = cur.shape[0]
    rowid = jax.lax.broadcasted_iota(jnp.int32, cur.shape, 0)
    xm1 = pltpu.roll(cur, shift=1, axis=0)
    xm1 = jnp.where(rowid == 0, 0.0, xm1)
    xp1 = pltpu.roll(cur, shift=n - 1, axis=0)
    xp1 = jnp.where(rowid == n - 1, 0.0, xp1)
    y = (jnp.dot(xm1, w0_ref[...], preferred_element_type=jnp.float32)
         + jnp.dot(cur, w1t_ref[...], preferred_element_type=jnp.float32)
         + jnp.dot(xp1, w2t_ref[...], preferred_element_type=jnp.float32))
    y = jax.nn.relu(y + bt_ref[...])
    z = jnp.dot(y, w1_ref[...], preferred_element_type=jnp.float32)
    z = jax.nn.relu(z + b1_ref[...])
    wf = jnp.dot(w2_ref[...], wfc_ref[...], preferred_element_type=jnp.float32)
    z = jnp.dot(z, wf, preferred_element_type=jnp.float32) + bf_ref[...]
    out_ref[...] = z


def kernel(x, edge_index, edge_weights, W_t, b_t, W1, b1, W2, b2, W_fc, b_fc):
    B, T, N, C = x.shape
    H = W1.shape[0]
    C_OUT = W_fc.shape[1]
    rows = B * N

    Wt0 = W_t[:, :, 0, 0].T
    Wt1 = W_t[:, :, 1, 0].T
    Wt2 = W_t[:, :, 2, 0].T
    bf = (b2 @ W_fc + b_fc).reshape(1, C_OUT)

    xl = x[:, T - 1].reshape(rows, C)

    out = pl.pallas_call(
        _chain_kernel,
        grid=(B,),
        in_specs=[
            pl.BlockSpec((N, C), lambda k: (k, 0)),
            pl.BlockSpec((C, H), lambda k: (0, 0)),
            pl.BlockSpec((C, H), lambda k: (0, 0)),
            pl.BlockSpec((C, H), lambda k: (0, 0)),
            pl.BlockSpec((H, H), lambda k: (0, 0)),
            pl.BlockSpec((H, H), lambda k: (0, 0)),
            pl.BlockSpec((H, C_OUT), lambda k: (0, 0)),
            pl.BlockSpec((1, H), lambda k: (0, 0)),
            pl.BlockSpec((1, H), lambda k: (0, 0)),
            pl.BlockSpec((1, C_OUT), lambda k: (0, 0)),
        ],
        out_specs=pl.BlockSpec((N, C_OUT), lambda k: (k, 0)),
        out_shape=jax.ShapeDtypeStruct((rows, C_OUT), jnp.float32),
    )(xl, Wt0, Wt1, Wt2, W1, W2, W_fc,
      b_t.reshape(1, H), b1.reshape(1, H), bf)
    return out.reshape(B, N, C_OUT)
